# Initial kernel scaffold; baseline (speedup 1.0000x reference)
import jax
import jax.numpy as jnp
from jax.experimental import pallas as pl


def kernel(x, edge_index, batch, flag, conv1_W0, conv1_W1, conv1_b, conv2_W0, conv2_W1, conv2_b, lin01_W, lin01_b, lin02_W, lin02_b, lin03_W, lin03_b, lin11_W, lin11_b, lin12_W, lin12_b, lin13_W, lin13_b):
    raise NotImplementedError("write your pallas kernel here")


# speedup vs baseline: 3.5082x; 3.5082x over previous
import functools

import jax
import jax.numpy as jnp
from jax import lax
from jax.experimental import pallas as pl
from jax.experimental.pallas import tpu as pltpu
from jax.experimental.pallas import tpu_sc as plsc

N = 10000
E = 160000
G = 64
F_H = 512
NC = 2
NS = 16
NW = NC * NS
EB = 128
NB = 40
E_PAD = NW * NB * EB
EPT = NB * EB
NB2 = 80
N_PAD = 10240
NPT = N_PAD // NS
CHUNK = 128
NCH = F_H // CHUNK
MB = 1000

@functools.cache
def _build_sc_deg():
    return functools.partial(
        pl.kernel,
        out_type=jax.ShapeDtypeStruct((NC, N_PAD), jnp.float32),
        mesh=plsc.VectorSubcoreMesh(core_axis_name="c", subcore_axis_name="s"),
        compiler_params=pltpu.CompilerParams(needs_layout_passes=False),
        scratch_types=[
            pltpu.VMEM((NB, EB), jnp.int32),
            pltpu.VMEM((N_PAD,), jnp.float32),
            pltpu.VMEM((NS, NPT), jnp.float32),
            pltpu.VMEM((NPT,), jnp.float32),
            pltpu.VMEM_SHARED((NS, N_PAD), jnp.float32),
        ],
    )(_sc_deg_body)


def _sc_deg(dst_p):
    return _build_sc_deg()(dst_p)


def _sc_deg_body(dst_hbm, deg_hbm, dstv, histv, colbuf, degv, part):
    c = lax.axis_index("c")
    s = lax.axis_index("s")
    wid = c * NS + s
    pltpu.sync_copy(dst_hbm.at[wid], dstv)

    def zbody(i, _):
        histv[pl.ds(i * 16, 16)] = jnp.zeros((16,), jnp.float32)
        return 0

    lax.fori_loop(0, N_PAD // 16, zbody, 0, unroll=4)

    ones = jnp.ones((16,), jnp.float32)
    lanes = lax.iota(jnp.int32, 16)
    base = wid * EPT

    def ebody(i, _):
        j = i // (EB // 16)
        k = i % (EB // 16)
        idx = dstv[j, pl.ds(k * 16, 16)]
        gid = base + i * 16 + lanes
        plsc.addupdate_scatter(histv, [idx], ones, mask=gid < E)
        return 0

    lax.fori_loop(0, EPT // 16, ebody, 0, unroll=4)

    pltpu.sync_copy(histv, part.at[s])
    plsc.subcore_barrier()
    pltpu.sync_copy(part.at[:, pl.ds(s * NPT, NPT)], colbuf)

    def sbody(i, _):
        acc = colbuf[0, pl.ds(i * 16, 16)]
        for j in range(1, NS):
            acc = acc + colbuf[j, pl.ds(i * 16, 16)]
        degv[pl.ds(i * 16, 16)] = acc
        return 0

    lax.fori_loop(0, NPT // 16, sbody, 0, unroll=2)
    pltpu.sync_copy(degv, deg_hbm.at[c, pl.ds(s * NPT, NPT)])


@functools.cache
def _build_sc_norm():
    return functools.partial(
        pl.kernel,
        out_type=jax.ShapeDtypeStruct((NW, NB, EB), jnp.float32),
        mesh=plsc.VectorSubcoreMesh(core_axis_name="c", subcore_axis_name="s"),
        compiler_params=pltpu.CompilerParams(needs_layout_passes=False),
        scratch_types=[
            pltpu.VMEM((N_PAD,), jnp.float32),
            pltpu.VMEM((NB, EB), jnp.int32),
            pltpu.VMEM((NB, EB), jnp.int32),
            pltpu.VMEM((NB, EB), jnp.float32),
        ],
    )(_sc_norm_body)


def _sc_norm(dinv, src_p, dst_p):
    return _build_sc_norm()(dinv, src_p, dst_p)


def _sc_norm_body(dinv_hbm, src_hbm, dst_hbm, norm_hbm,
                  dinvv, srcv, dstv, normv):
    c = lax.axis_index("c")
    s = lax.axis_index("s")
    wid = c * NS + s
    pltpu.sync_copy(dinv_hbm.at[pl.ds(0, N_PAD)], dinvv)
    pltpu.sync_copy(src_hbm.at[wid], srcv)
    pltpu.sync_copy(dst_hbm.at[wid], dstv)

    lanes = lax.iota(jnp.int32, 16)
    base = wid * EPT

    def ebody(i, _):
        j = i // (EB // 16)
        k = i % (EB // 16)
        sidx = srcv[j, pl.ds(k * 16, 16)]
        didx = dstv[j, pl.ds(k * 16, 16)]
        a = plsc.load_gather(dinvv, [sidx])
        b = plsc.load_gather(dinvv, [didx])
        gid = base + i * 16 + lanes
        nv = jnp.where(gid < E, -(a * b), jnp.zeros((16,), jnp.float32))
        normv[j, pl.ds(k * 16, 16)] = nv
        return 0

    lax.fori_loop(0, EPT // 16, ebody, 0, unroll=4)
    pltpu.sync_copy(normv, norm_hbm.at[wid])


@functools.cache
def _build_sc_prop():
    return functools.partial(
        pl.kernel,
        out_type=jax.ShapeDtypeStruct((NCH * N_PAD, CHUNK), jnp.float32),
        mesh=plsc.VectorSubcoreMesh(core_axis_name="c", subcore_axis_name="s"),
        compiler_params=pltpu.CompilerParams(needs_layout_passes=False),
        scratch_types=[
            pltpu.VMEM((NB2, EB), jnp.int32),
            pltpu.VMEM((NB2, EB), jnp.int32),
            pltpu.VMEM((NB2, EB), jnp.float32),
            pltpu.VMEM((EB, CHUNK), jnp.float32),
            pltpu.VMEM_SHARED((N_PAD, CHUNK), jnp.float32),
            pltpu.SemaphoreType.DMA,
        ],
    )(_sc_prop_body)


def _sc_prop(y2, src_p, dst_p, norm):
    return _build_sc_prop()(
        y2,
        src_p.reshape(NS, NB2, EB),
        dst_p.reshape(NS, NB2, EB),
        norm.reshape(NS, NB2, EB),
    )


def _sc_prop_body(y2_hbm, src_hbm, dst_hbm, norm_hbm, tx_hbm,
                  srcv2, dstv, normv, rows, acc, gsem):
    c = lax.axis_index("c")
    s = lax.axis_index("s")
    pltpu.sync_copy(dst_hbm.at[s], dstv)
    pltpu.sync_copy(norm_hbm.at[s], normv)

    for ch in range(NCH // NC):
        cidx = c * (NCH // NC) + ch

        pltpu.sync_copy(src_hbm.at[s], srcv2)
        off = cidx * N

        def obody(i, _):
            j = i // (EB // 16)
            k = i % (EB // 16)
            srcv2[j, pl.ds(k * 16, 16)] = srcv2[j, pl.ds(k * 16, 16)] + off
            return 0

        lax.fori_loop(0, NB2 * EB // 16, obody, 0, unroll=4)

        def zbody(i, _):
            for k in range(CHUNK // 16):
                rows[i, pl.ds(k * 16, 16)] = jnp.zeros((16,), jnp.float32)
            return 0

        lax.fori_loop(0, EB, zbody, 0, unroll=2)
        for t in range(NPT // EB):
            pltpu.sync_copy(rows, acc.at[pl.ds(s * NPT + t * EB, EB)])
        plsc.subcore_barrier()

        def bbody(j, _):
            pltpu.async_copy(y2_hbm.at[srcv2.at[j]], rows, gsem).wait()

            def rbody(rg, _):
                nvec = normv[j, pl.ds(rg * 16, 16)]
                for i in range(16):
                    sv = jnp.full((16,), nvec[i], jnp.float32)
                    r = rg * 16 + i
                    for k in range(CHUNK // 16):
                        rows[r, pl.ds(k * 16, 16)] = (
                            rows[r, pl.ds(k * 16, 16)] * sv)
                return 0

            lax.fori_loop(0, EB // 16, rbody, 0)
            pltpu.sync_copy(rows, acc.at[dstv.at[j]], add=True)
            return 0

        lax.fori_loop(0, NB2, bbody, 0)
        plsc.subcore_barrier()

        for t in range(NPT // EB):
            pltpu.sync_copy(acc.at[pl.ds(s * NPT + t * EB, EB)], rows)
            pltpu.sync_copy(
                rows, tx_hbm.at[pl.ds(cidx * N_PAD + s * NPT + t * EB, EB)])
        plsc.subcore_barrier()


def _dinv_body(deg_ref, dinv_ref):
    a = deg_ref[...]
    d = a[0:1, :] + a[1:2, :]
    dinv_ref[...] = jnp.where(d > 0, lax.rsqrt(jnp.maximum(d, 1e-12)), 0.0)


def _tc_dinv(deg):
    return pl.pallas_call(
        _dinv_body,
        out_shape=jax.ShapeDtypeStruct((1, N_PAD), jnp.float32),
    )(deg)


def _mm_body(x_ref, w_ref, o_ref):
    o_ref[...] = jnp.dot(x_ref[...], w_ref[...],
                         preferred_element_type=jnp.float32)


def _tc_matmul(x, w):
    k = x.shape[1]
    return pl.pallas_call(
        _mm_body,
        grid=(N // MB,),
        in_specs=[
            pl.BlockSpec((MB, k), lambda i: (i, 0)),
            pl.BlockSpec((k, F_H), lambda i: (0, 0)),
        ],
        out_specs=pl.BlockSpec((MB, F_H), lambda i: (i, 0)),
        out_shape=jax.ShapeDtypeStruct((N, F_H), jnp.float32),
    )(x, w)


def _mmc_body(x_ref, w_ref, o_ref):
    o_ref[...] = jnp.dot(x_ref[...], w_ref[...],
                         preferred_element_type=jnp.float32)[None]


def _tc_matmul_chunked(x, w):
    k = x.shape[1]
    return pl.pallas_call(
        _mmc_body,
        grid=(N // MB, NCH),
        in_specs=[
            pl.BlockSpec((MB, k), lambda i, j: (i, 0)),
            pl.BlockSpec((k, CHUNK), lambda i, j: (0, j)),
        ],
        out_specs=pl.BlockSpec((1, MB, CHUNK), lambda i, j: (j, i, 0)),
        out_shape=jax.ShapeDtypeStruct((NCH, N, CHUNK), jnp.float32),
    )(x, w)


def _relu_body(xw0_ref, tx_ref, b_ref, o_ref):
    t = tx_ref[...]
    tcat = jnp.concatenate([t[0], t[1], t[2], t[3]], axis=-1)
    o_ref[...] = jnp.maximum(xw0_ref[...] + tcat + b_ref[...], 0.0)


def _tc_add_relu(xw0, tx, b):
    return pl.pallas_call(
        _relu_body,
        grid=(N // MB,),
        in_specs=[
            pl.BlockSpec((MB, F_H), lambda i: (i, 0)),
            pl.BlockSpec((NCH, MB, CHUNK), lambda i: (0, i, 0)),
            pl.BlockSpec((1, F_H), lambda i: (0, 0)),
        ],
        out_specs=pl.BlockSpec((MB, F_H), lambda i: (i, 0)),
        out_shape=jax.ShapeDtypeStruct((N, F_H), jnp.float32),
    )(xw0, tx, b)


def _pool_body(h_ref, btr_ref, btc_ref, gmp_ref, gsum_ref, cnt_ref, m_ref):
    i = pl.program_id(0)
    h2 = h_ref[...]
    btr = btr_ref[0]
    btc = btc_ref[...]

    gids = lax.broadcasted_iota(jnp.int32, (G, MB), 0)
    m = (gids == jnp.broadcast_to(btr, (G, MB))).astype(jnp.float32)

    gids_t = lax.broadcasted_iota(jnp.int32, (MB, G), 1)
    m_ref[...] = (jnp.broadcast_to(btc, (MB, G)) == gids_t).astype(jnp.float32)

    bsum = jnp.dot(m, h2, preferred_element_type=jnp.float32)
    bcnt = jnp.sum(m, axis=1, keepdims=True)

    def mbody(k, acc):
        hblk = h_ref[pl.ds(k * 8, 8), :]
        mblk = m_ref[pl.ds(k * 8, 8), :]
        w = jnp.where(mblk[:, :, None] > 0, hblk[:, None, :], -3e38)
        return jnp.maximum(acc, jnp.max(w, axis=0))

    bmax = lax.fori_loop(0, MB // 8, mbody,
                         jnp.full((G, F_H), -3e38, jnp.float32))

    @pl.when(i == 0)
    def _():
        gmp_ref[...] = jnp.full((G, F_H), -3e38, jnp.float32)
        gsum_ref[...] = jnp.zeros((G, F_H), jnp.float32)
        cnt_ref[...] = jnp.zeros((G, 1), jnp.float32)

    gmp_ref[...] = jnp.maximum(gmp_ref[...], bmax)
    gsum_ref[...] = gsum_ref[...] + bsum
    cnt_ref[...] = cnt_ref[...] + bcnt


def _tc_pool(h2, batch_row3, batch_col):
    return pl.pallas_call(
        _pool_body,
        grid=(N // MB,),
        in_specs=[
            pl.BlockSpec((MB, F_H), lambda i: (i, 0)),
            pl.BlockSpec((1, 1, MB), lambda i: (i, 0, 0)),
            pl.BlockSpec((MB, 1), lambda i: (i, 0)),
        ],
        out_specs=[
            pl.BlockSpec((G, F_H), lambda i: (0, 0)),
            pl.BlockSpec((G, F_H), lambda i: (0, 0)),
            pl.BlockSpec((G, 1), lambda i: (0, 0)),
        ],
        out_shape=[
            jax.ShapeDtypeStruct((G, F_H), jnp.float32),
            jax.ShapeDtypeStruct((G, F_H), jnp.float32),
            jax.ShapeDtypeStruct((G, 1), jnp.float32),
        ],
        scratch_shapes=[pltpu.VMEM((MB, G), jnp.float32)],
    )(h2, batch_row3, batch_col)


def _softmax(x):
    e = jnp.exp(x - jnp.max(x, axis=-1, keepdims=True))
    return e / jnp.sum(e, axis=-1, keepdims=True)


def _heads_body(gmp_ref, gsum_ref, cnt_ref,
                w01_ref, b01_ref, w02_ref, b02_ref, w03_ref, b03_ref,
                w11_ref, b11_ref, w12_ref, b12_ref, w13_ref, b13_ref,
                out0_ref, out1_ref, feat0_ref, feat1_ref):
    cnt = cnt_ref[...]
    gap = gsum_ref[...] / jnp.maximum(cnt, 1.0)
    gmp = jnp.where(cnt > 0, gmp_ref[...], 0.0)
    feat = jnp.concatenate([gmp, gap], axis=1)
    feat0_ref[...] = feat

    def ff(v, w_ref, b_ref):
        return jnp.maximum(
            jnp.dot(v, w_ref[...], preferred_element_type=jnp.float32)
            + b_ref[...], 0.0)

    x0 = ff(feat, w01_ref, b01_ref)
    x0 = ff(x0, w02_ref, b02_ref)
    x0 = ff(x0, w03_ref, b03_ref)
    out0_ref[...] = _softmax(x0)

    x1 = ff(feat, w11_ref, b11_ref)
    x1 = ff(x1, w12_ref, b12_ref)
    feat1_ref[...] = x1
    x1 = ff(x1, w13_ref, b13_ref)
    out1_ref[...] = _softmax(x1)


def _tc_heads(gmp, gsum, cnt, w01, b01, w02, b02, w03, b03,
              w11, b11, w12, b12, w13, b13):
    return pl.pallas_call(
        _heads_body,
        out_shape=[
            jax.ShapeDtypeStruct((G, 2), jnp.float32),
            jax.ShapeDtypeStruct((G, 4), jnp.float32),
            jax.ShapeDtypeStruct((G, 1024), jnp.float32),
            jax.ShapeDtypeStruct((G, 256), jnp.float32),
        ],
    )(gmp, gsum, cnt, w01, b01, w02, b02, w03, b03,
      w11, b11, w12, b12, w13, b13)


def kernel(x, edge_index, batch, flag,
           conv1_W0, conv1_W1, conv1_b, conv2_W0, conv2_W1, conv2_b,
           lin01_W, lin01_b, lin02_W, lin02_b, lin03_W, lin03_b,
           lin11_W, lin11_b, lin12_W, lin12_b, lin13_W, lin13_b):
    src = edge_index[0].astype(jnp.int32)
    dst = edge_index[1].astype(jnp.int32)
    src_p = jnp.zeros((E_PAD,), jnp.int32).at[:E].set(src).reshape(NW, NB, EB)
    dst_p = jnp.zeros((E_PAD,), jnp.int32).at[:E].set(dst).reshape(NW, NB, EB)

    batch_i = batch.astype(jnp.int32)
    batch_row3 = batch_i.reshape(N // MB, 1, MB)
    batch_col = batch_i.reshape(N, 1)

    deg = _sc_deg(dst_p)
    dinv = _tc_dinv(deg)
    norm = _sc_norm(dinv.reshape(N_PAD), src_p, dst_p)

    xw0 = _tc_matmul(x, conv1_W0)
    xw1c = _tc_matmul_chunked(x, conv1_W1).reshape(NCH * N, CHUNK)
    tx1 = _sc_prop(xw1c, src_p, dst_p, norm).reshape(NCH, N_PAD, CHUNK)
    h1 = _tc_add_relu(xw0, tx1[:, :N, :], conv1_b.reshape(1, F_H))

    hw0 = _tc_matmul(h1, conv2_W0)
    hw1c = _tc_matmul_chunked(h1, conv2_W1).reshape(NCH * N, CHUNK)
    tx2 = _sc_prop(hw1c, src_p, dst_p, norm).reshape(NCH, N_PAD, CHUNK)
    h2 = _tc_add_relu(hw0, tx2[:, :N, :], conv2_b.reshape(1, F_H))

    gmp, gsum, cnt = _tc_pool(h2, batch_row3, batch_col)
    out0, out1, feat0, feat1 = _tc_heads(
        gmp, gsum, cnt,
        lin01_W, lin01_b.reshape(1, F_H), lin02_W, lin02_b.reshape(1, 256),
        lin03_W, lin03_b.reshape(1, 2),
        lin11_W, lin11_b.reshape(1, F_H), lin12_W, lin12_b.reshape(1, 256),
        lin13_W, lin13_b.reshape(1, 4))
    return (out0, out1, feat0, feat1)


# speedup vs baseline: 4.0839x; 1.1641x over previous
import functools

import jax
import jax.numpy as jnp
from jax import lax
from jax.experimental import pallas as pl
from jax.experimental.pallas import tpu as pltpu
from jax.experimental.pallas import tpu_sc as plsc

N = 10000
E = 160000
G = 64
F_H = 512
NC = 2
NS = 16
NW = NC * NS
EB = 128
NB = 40
E_PAD = NW * NB * EB
EPT = NB * EB
NB2 = 80
NBH = 40
N_PAD = 10240
NPT = N_PAD // NS
CHUNK = 128
NCH = F_H // CHUNK
MB = 1000

@functools.cache
def _build_sc_deg():
    return functools.partial(
        pl.kernel,
        out_type=jax.ShapeDtypeStruct((NC, N_PAD), jnp.float32),
        mesh=plsc.VectorSubcoreMesh(core_axis_name="c", subcore_axis_name="s"),
        compiler_params=pltpu.CompilerParams(needs_layout_passes=False),
        scratch_types=[
            pltpu.VMEM((NB, EB), jnp.int32),
            pltpu.VMEM((N_PAD,), jnp.float32),
            pltpu.VMEM((NS, NPT), jnp.float32),
            pltpu.VMEM((NPT,), jnp.float32),
            pltpu.VMEM_SHARED((NS, N_PAD), jnp.float32),
        ],
    )(_sc_deg_body)


def _sc_deg(dst_p):
    return _build_sc_deg()(dst_p)


def _sc_deg_body(dst_hbm, deg_hbm, dstv, histv, colbuf, degv, part):
    c = lax.axis_index("c")
    s = lax.axis_index("s")
    wid = c * NS + s
    pltpu.sync_copy(dst_hbm.at[wid], dstv)

    def zbody(i, _):
        histv[pl.ds(i * 16, 16)] = jnp.zeros((16,), jnp.float32)
        return 0

    lax.fori_loop(0, N_PAD // 16, zbody, 0, unroll=4)

    ones = jnp.ones((16,), jnp.float32)
    lanes = lax.iota(jnp.int32, 16)
    base = wid * EPT

    def ebody(i, _):
        j = i // (EB // 16)
        k = i % (EB // 16)
        idx = dstv[j, pl.ds(k * 16, 16)]
        gid = base + i * 16 + lanes
        plsc.addupdate_scatter(histv, [idx], ones, mask=gid < E)
        return 0

    lax.fori_loop(0, EPT // 16, ebody, 0, unroll=4)

    pltpu.sync_copy(histv, part.at[s])
    plsc.subcore_barrier()
    pltpu.sync_copy(part.at[:, pl.ds(s * NPT, NPT)], colbuf)

    def sbody(i, _):
        acc = colbuf[0, pl.ds(i * 16, 16)]
        for j in range(1, NS):
            acc = acc + colbuf[j, pl.ds(i * 16, 16)]
        degv[pl.ds(i * 16, 16)] = acc
        return 0

    lax.fori_loop(0, NPT // 16, sbody, 0, unroll=2)
    pltpu.sync_copy(degv, deg_hbm.at[c, pl.ds(s * NPT, NPT)])


@functools.cache
def _build_sc_norm():
    return functools.partial(
        pl.kernel,
        out_type=jax.ShapeDtypeStruct((NW, NB, EB), jnp.float32),
        mesh=plsc.VectorSubcoreMesh(core_axis_name="c", subcore_axis_name="s"),
        compiler_params=pltpu.CompilerParams(needs_layout_passes=False),
        scratch_types=[
            pltpu.VMEM((N_PAD,), jnp.float32),
            pltpu.VMEM((NB, EB), jnp.int32),
            pltpu.VMEM((NB, EB), jnp.int32),
            pltpu.VMEM((NB, EB), jnp.float32),
        ],
    )(_sc_norm_body)


def _sc_norm(dinv, src_p, dst_p):
    return _build_sc_norm()(dinv, src_p, dst_p)


def _sc_norm_body(dinv_hbm, src_hbm, dst_hbm, norm_hbm,
                  dinvv, srcv, dstv, normv):
    c = lax.axis_index("c")
    s = lax.axis_index("s")
    wid = c * NS + s
    pltpu.sync_copy(dinv_hbm.at[pl.ds(0, N_PAD)], dinvv)
    pltpu.sync_copy(src_hbm.at[wid], srcv)
    pltpu.sync_copy(dst_hbm.at[wid], dstv)

    lanes = lax.iota(jnp.int32, 16)
    base = wid * EPT

    def ebody(i, _):
        j = i // (EB // 16)
        k = i % (EB // 16)
        sidx = srcv[j, pl.ds(k * 16, 16)]
        didx = dstv[j, pl.ds(k * 16, 16)]
        a = plsc.load_gather(dinvv, [sidx])
        b = plsc.load_gather(dinvv, [didx])
        gid = base + i * 16 + lanes
        nv = jnp.where(gid < E, -(a * b), jnp.zeros((16,), jnp.float32))
        normv[j, pl.ds(k * 16, 16)] = nv
        return 0

    lax.fori_loop(0, EPT // 16, ebody, 0, unroll=4)
    pltpu.sync_copy(normv, norm_hbm.at[wid])


@functools.cache
def _build_sc_prop():
    return functools.partial(
        pl.kernel,
        out_type=jax.ShapeDtypeStruct((NCH * N_PAD, CHUNK), jnp.float32),
        mesh=plsc.VectorSubcoreMesh(core_axis_name="c", subcore_axis_name="s"),
        compiler_params=pltpu.CompilerParams(needs_layout_passes=False),
        scratch_types=[
            pltpu.VMEM((NBH, EB), jnp.int32),
            pltpu.VMEM((NBH, EB), jnp.int32),
            pltpu.VMEM((NBH, EB), jnp.float32),
            pltpu.VMEM((EB, CHUNK), jnp.float32),
            pltpu.VMEM((EB, CHUNK), jnp.float32),
            pltpu.VMEM_SHARED((N_PAD, CHUNK), jnp.float32),
            pltpu.SemaphoreType.DMA,
            pltpu.SemaphoreType.DMA,
            pltpu.SemaphoreType.DMA,
            pltpu.SemaphoreType.DMA,
        ],
    )(_sc_prop_body)


def _sc_prop(y2, src_p, dst_p, norm):
    return _build_sc_prop()(
        y2,
        src_p.reshape(NS, NB2, EB),
        dst_p.reshape(NS, NB2, EB),
        norm.reshape(NS, NB2, EB),
    )


def _sc_prop_body(y2_hbm, src_hbm, dst_hbm, norm_hbm, tx_hbm,
                  srcv2, dstv, normv, b0, b1, acc, gs0, gs1, ss0, ss1):
    c = lax.axis_index("c")
    s = lax.axis_index("s")

    def scale(buf, j, jv=None):
        def rbody(rg, _):
            nvec = normv[j, pl.ds(rg * 16, 16)]
            for i in range(16):
                sv = jnp.full((16,), nvec[i], jnp.float32)
                r = rg * 16 + i
                for k in range(CHUNK // 16):
                    buf[r, pl.ds(k * 16, 16)] = buf[r, pl.ds(k * 16, 16)] * sv
            return 0

        lax.fori_loop(0, EB // 16, rbody, 0)

    def gather(j, buf, sem):
        return pltpu.async_copy(y2_hbm.at[srcv2.at[j]], buf, sem)

    def scatter(j, buf, sem):
        return pltpu.async_copy(buf, acc.at[dstv.at[j]], sem, add=True)

    for ch in range(NCH // NC):
        cidx = c * (NCH // NC) + ch
        off = cidx * N

        def zbody(i, _):
            for k in range(CHUNK // 16):
                b0[i, pl.ds(k * 16, 16)] = jnp.zeros((16,), jnp.float32)
            return 0

        lax.fori_loop(0, EB, zbody, 0, unroll=2)
        for t in range(NPT // EB):
            pltpu.sync_copy(b0, acc.at[pl.ds(s * NPT + t * EB, EB)])
        plsc.subcore_barrier()

        for half in range(NB2 // NBH):
            pltpu.sync_copy(src_hbm.at[s, pl.ds(half * NBH, NBH)], srcv2)
            pltpu.sync_copy(dst_hbm.at[s, pl.ds(half * NBH, NBH)], dstv)
            pltpu.sync_copy(norm_hbm.at[s, pl.ds(half * NBH, NBH)], normv)

            def obody(i, _):
                j = i // (EB // 16)
                k = i % (EB // 16)
                srcv2[j, pl.ds(k * 16, 16)] = (
                    srcv2[j, pl.ds(k * 16, 16)] + off)
                return 0

            lax.fori_loop(0, NBH * EB // 16, obody, 0, unroll=4)

            gather(0, b0, gs0)

            def pair(j2, _):
                j = j2 * 2

                @pl.when(j2 > 0)
                def _():
                    pltpu.make_async_copy(b1, acc.at[dstv.at[0]], ss1).wait()

                g1 = gather(j + 1, b1, gs1)
                pltpu.make_async_copy(y2_hbm.at[srcv2.at[0]], b0, gs0).wait()
                scale(b0, j)
                s0 = scatter(j, b0, ss0)
                g1.wait()
                scale(b1, j + 1)
                scatter(j + 1, b1, ss1)
                s0.wait()

                @pl.when(j2 < NBH // 2 - 1)
                def _():
                    gather(j + 2, b0, gs0)

                return 0

            lax.fori_loop(0, NBH // 2, pair, 0)
            pltpu.make_async_copy(b1, acc.at[dstv.at[0]], ss1).wait()

        plsc.subcore_barrier()

        for t in range(NPT // EB):
            pltpu.sync_copy(acc.at[pl.ds(s * NPT + t * EB, EB)], b0)
            pltpu.sync_copy(
                b0, tx_hbm.at[pl.ds(cidx * N_PAD + s * NPT + t * EB, EB)])
        plsc.subcore_barrier()


def _dinv_body(deg_ref, dinv_ref):
    a = deg_ref[...]
    d = a[0:1, :] + a[1:2, :]
    dinv_ref[...] = jnp.where(d > 0, lax.rsqrt(jnp.maximum(d, 1e-12)), 0.0)


def _tc_dinv(deg):
    return pl.pallas_call(
        _dinv_body,
        out_shape=jax.ShapeDtypeStruct((1, N_PAD), jnp.float32),
    )(deg)


def _mm_body(x_ref, w_ref, o_ref):
    o_ref[...] = jnp.dot(x_ref[...], w_ref[...],
                         preferred_element_type=jnp.float32)


def _tc_matmul(x, w):
    k = x.shape[1]
    return pl.pallas_call(
        _mm_body,
        grid=(N // MB,),
        in_specs=[
            pl.BlockSpec((MB, k), lambda i: (i, 0)),
            pl.BlockSpec((k, F_H), lambda i: (0, 0)),
        ],
        out_specs=pl.BlockSpec((MB, F_H), lambda i: (i, 0)),
        out_shape=jax.ShapeDtypeStruct((N, F_H), jnp.float32),
    )(x, w)


def _mmc_body(x_ref, w_ref, o_ref):
    o_ref[...] = jnp.dot(x_ref[...], w_ref[...],
                         preferred_element_type=jnp.float32)[None]


def _tc_matmul_chunked(x, w):
    k = x.shape[1]
    return pl.pallas_call(
        _mmc_body,
        grid=(N // MB, NCH),
        in_specs=[
            pl.BlockSpec((MB, k), lambda i, j: (i, 0)),
            pl.BlockSpec((k, CHUNK), lambda i, j: (0, j)),
        ],
        out_specs=pl.BlockSpec((1, MB, CHUNK), lambda i, j: (j, i, 0)),
        out_shape=jax.ShapeDtypeStruct((NCH, N, CHUNK), jnp.float32),
    )(x, w)


def _relu_body(xw0_ref, tx_ref, b_ref, o_ref):
    t = tx_ref[...]
    tcat = jnp.concatenate([t[0], t[1], t[2], t[3]], axis=-1)
    o_ref[...] = jnp.maximum(xw0_ref[...] + tcat + b_ref[...], 0.0)


def _tc_add_relu(xw0, tx, b):
    return pl.pallas_call(
        _relu_body,
        grid=(N // MB,),
        in_specs=[
            pl.BlockSpec((MB, F_H), lambda i: (i, 0)),
            pl.BlockSpec((NCH, MB, CHUNK), lambda i: (0, i, 0)),
            pl.BlockSpec((1, F_H), lambda i: (0, 0)),
        ],
        out_specs=pl.BlockSpec((MB, F_H), lambda i: (i, 0)),
        out_shape=jax.ShapeDtypeStruct((N, F_H), jnp.float32),
    )(xw0, tx, b)


def _pool_body(h_ref, btr_ref, btc_ref, gmp_ref, gsum_ref, cnt_ref, m_ref):
    i = pl.program_id(0)
    h2 = h_ref[...]
    btr = btr_ref[0]
    btc = btc_ref[...]

    gids = lax.broadcasted_iota(jnp.int32, (G, MB), 0)
    m = (gids == jnp.broadcast_to(btr, (G, MB))).astype(jnp.float32)

    gids_t = lax.broadcasted_iota(jnp.int32, (MB, G), 1)
    m_ref[...] = (jnp.broadcast_to(btc, (MB, G)) == gids_t).astype(jnp.float32)

    bsum = jnp.dot(m, h2, preferred_element_type=jnp.float32)
    bcnt = jnp.sum(m, axis=1, keepdims=True)

    def mbody(k, acc):
        hblk = h_ref[pl.ds(k * 8, 8), :]
        mblk = m_ref[pl.ds(k * 8, 8), :]
        w = jnp.where(mblk[:, :, None] > 0, hblk[:, None, :], -3e38)
        return jnp.maximum(acc, jnp.max(w, axis=0))

    bmax = lax.fori_loop(0, MB // 8, mbody,
                         jnp.full((G, F_H), -3e38, jnp.float32))

    @pl.when(i == 0)
    def _():
        gmp_ref[...] = jnp.full((G, F_H), -3e38, jnp.float32)
        gsum_ref[...] = jnp.zeros((G, F_H), jnp.float32)
        cnt_ref[...] = jnp.zeros((G, 1), jnp.float32)

    gmp_ref[...] = jnp.maximum(gmp_ref[...], bmax)
    gsum_ref[...] = gsum_ref[...] + bsum
    cnt_ref[...] = cnt_ref[...] + bcnt


def _tc_pool(h2, batch_row3, batch_col):
    return pl.pallas_call(
        _pool_body,
        grid=(N // MB,),
        in_specs=[
            pl.BlockSpec((MB, F_H), lambda i: (i, 0)),
            pl.BlockSpec((1, 1, MB), lambda i: (i, 0, 0)),
            pl.BlockSpec((MB, 1), lambda i: (i, 0)),
        ],
        out_specs=[
            pl.BlockSpec((G, F_H), lambda i: (0, 0)),
            pl.BlockSpec((G, F_H), lambda i: (0, 0)),
            pl.BlockSpec((G, 1), lambda i: (0, 0)),
        ],
        out_shape=[
            jax.ShapeDtypeStruct((G, F_H), jnp.float32),
            jax.ShapeDtypeStruct((G, F_H), jnp.float32),
            jax.ShapeDtypeStruct((G, 1), jnp.float32),
        ],
        scratch_shapes=[pltpu.VMEM((MB, G), jnp.float32)],
    )(h2, batch_row3, batch_col)


def _softmax(x):
    e = jnp.exp(x - jnp.max(x, axis=-1, keepdims=True))
    return e / jnp.sum(e, axis=-1, keepdims=True)


def _heads_body(gmp_ref, gsum_ref, cnt_ref,
                w01_ref, b01_ref, w02_ref, b02_ref, w03_ref, b03_ref,
                w11_ref, b11_ref, w12_ref, b12_ref, w13_ref, b13_ref,
                out0_ref, out1_ref, feat0_ref, feat1_ref):
    cnt = cnt_ref[...]
    gap = gsum_ref[...] / jnp.maximum(cnt, 1.0)
    gmp = jnp.where(cnt > 0, gmp_ref[...], 0.0)
    feat = jnp.concatenate([gmp, gap], axis=1)
    feat0_ref[...] = feat

    def ff(v, w_ref, b_ref):
        return jnp.maximum(
            jnp.dot(v, w_ref[...], preferred_element_type=jnp.float32)
            + b_ref[...], 0.0)

    x0 = ff(feat, w01_ref, b01_ref)
    x0 = ff(x0, w02_ref, b02_ref)
    x0 = ff(x0, w03_ref, b03_ref)
    out0_ref[...] = _softmax(x0)

    x1 = ff(feat, w11_ref, b11_ref)
    x1 = ff(x1, w12_ref, b12_ref)
    feat1_ref[...] = x1
    x1 = ff(x1, w13_ref, b13_ref)
    out1_ref[...] = _softmax(x1)


def _tc_heads(gmp, gsum, cnt, w01, b01, w02, b02, w03, b03,
              w11, b11, w12, b12, w13, b13):
    return pl.pallas_call(
        _heads_body,
        out_shape=[
            jax.ShapeDtypeStruct((G, 2), jnp.float32),
            jax.ShapeDtypeStruct((G, 4), jnp.float32),
            jax.ShapeDtypeStruct((G, 1024), jnp.float32),
            jax.ShapeDtypeStruct((G, 256), jnp.float32),
        ],
    )(gmp, gsum, cnt, w01, b01, w02, b02, w03, b03,
      w11, b11, w12, b12, w13, b13)


def kernel(x, edge_index, batch, flag,
           conv1_W0, conv1_W1, conv1_b, conv2_W0, conv2_W1, conv2_b,
           lin01_W, lin01_b, lin02_W, lin02_b, lin03_W, lin03_b,
           lin11_W, lin11_b, lin12_W, lin12_b, lin13_W, lin13_b):
    src = edge_index[0].astype(jnp.int32)
    dst = edge_index[1].astype(jnp.int32)
    src_p = jnp.zeros((E_PAD,), jnp.int32).at[:E].set(src).reshape(NW, NB, EB)
    dst_p = jnp.zeros((E_PAD,), jnp.int32).at[:E].set(dst).reshape(NW, NB, EB)

    batch_i = batch.astype(jnp.int32)
    batch_row3 = batch_i.reshape(N // MB, 1, MB)
    batch_col = batch_i.reshape(N, 1)

    deg = _sc_deg(dst_p)
    dinv = _tc_dinv(deg)
    norm = _sc_norm(dinv.reshape(N_PAD), src_p, dst_p)

    xw0 = _tc_matmul(x, conv1_W0)
    xw1c = _tc_matmul_chunked(x, conv1_W1).reshape(NCH * N, CHUNK)
    tx1 = _sc_prop(xw1c, src_p, dst_p, norm).reshape(NCH, N_PAD, CHUNK)
    h1 = _tc_add_relu(xw0, tx1[:, :N, :], conv1_b.reshape(1, F_H))

    hw0 = _tc_matmul(h1, conv2_W0)
    hw1c = _tc_matmul_chunked(h1, conv2_W1).reshape(NCH * N, CHUNK)
    tx2 = _sc_prop(hw1c, src_p, dst_p, norm).reshape(NCH, N_PAD, CHUNK)
    h2 = _tc_add_relu(hw0, tx2[:, :N, :], conv2_b.reshape(1, F_H))

    gmp, gsum, cnt = _tc_pool(h2, batch_row3, batch_col)
    out0, out1, feat0, feat1 = _tc_heads(
        gmp, gsum, cnt,
        lin01_W, lin01_b.reshape(1, F_H), lin02_W, lin02_b.reshape(1, 256),
        lin03_W, lin03_b.reshape(1, 2),
        lin11_W, lin11_b.reshape(1, F_H), lin12_W, lin12_b.reshape(1, 256),
        lin13_W, lin13_b.reshape(1, 4))
    return (out0, out1, feat0, feat1)


# speedup vs baseline: 4.2081x; 1.0304x over previous
import functools

import jax
import jax.numpy as jnp
from jax import lax
from jax.experimental import pallas as pl
from jax.experimental.pallas import tpu as pltpu
from jax.experimental.pallas import tpu_sc as plsc

N = 10000
E = 160000
G = 64
F_H = 512
NC = 2
NS = 16
NW = NC * NS
EB = 128
NB = 40
E_PAD = NW * NB * EB
EPT = NB * EB
NB2 = 80
NBH = 40
N_PAD = 10240
NPT = N_PAD // NS
CHUNK = 128
NCH = F_H // CHUNK
MB = 1000

@functools.cache
def _build_sc_deg():
    return functools.partial(
        pl.kernel,
        out_type=jax.ShapeDtypeStruct((NC, N_PAD), jnp.float32),
        mesh=plsc.VectorSubcoreMesh(core_axis_name="c", subcore_axis_name="s"),
        compiler_params=pltpu.CompilerParams(needs_layout_passes=False),
        scratch_types=[
            pltpu.VMEM((NB, EB), jnp.int32),
            pltpu.VMEM((N_PAD,), jnp.float32),
            pltpu.VMEM((NS, NPT), jnp.float32),
            pltpu.VMEM((NPT,), jnp.float32),
            pltpu.VMEM_SHARED((NS, N_PAD), jnp.float32),
        ],
    )(_sc_deg_body)


def _sc_deg(dst_p):
    return _build_sc_deg()(dst_p)


def _sc_deg_body(dst_hbm, deg_hbm, dstv, histv, colbuf, degv, part):
    c = lax.axis_index("c")
    s = lax.axis_index("s")
    wid = c * NS + s
    pltpu.sync_copy(dst_hbm.at[wid], dstv)

    def zbody(i, _):
        histv[pl.ds(i * 16, 16)] = jnp.zeros((16,), jnp.float32)
        return 0

    lax.fori_loop(0, N_PAD // 16, zbody, 0, unroll=4)

    ones = jnp.ones((16,), jnp.float32)
    lanes = lax.iota(jnp.int32, 16)
    base = wid * EPT

    def ebody(i, _):
        j = i // (EB // 16)
        k = i % (EB // 16)
        idx = dstv[j, pl.ds(k * 16, 16)]
        gid = base + i * 16 + lanes
        plsc.addupdate_scatter(histv, [idx], ones, mask=gid < E)
        return 0

    lax.fori_loop(0, EPT // 16, ebody, 0, unroll=4)

    pltpu.sync_copy(histv, part.at[s])
    plsc.subcore_barrier()
    pltpu.sync_copy(part.at[:, pl.ds(s * NPT, NPT)], colbuf)

    def sbody(i, _):
        acc = colbuf[0, pl.ds(i * 16, 16)]
        for j in range(1, NS):
            acc = acc + colbuf[j, pl.ds(i * 16, 16)]
        degv[pl.ds(i * 16, 16)] = acc
        return 0

    lax.fori_loop(0, NPT // 16, sbody, 0, unroll=2)
    pltpu.sync_copy(degv, deg_hbm.at[c, pl.ds(s * NPT, NPT)])


@functools.cache
def _build_sc_prop():
    return functools.partial(
        pl.kernel,
        out_type=jax.ShapeDtypeStruct((NCH * N_PAD, CHUNK), jnp.float32),
        mesh=plsc.VectorSubcoreMesh(core_axis_name="c", subcore_axis_name="s"),
        compiler_params=pltpu.CompilerParams(needs_layout_passes=False),
        scratch_types=[
            pltpu.VMEM((NBH, EB), jnp.int32),
            pltpu.VMEM((NBH, EB), jnp.int32),
            pltpu.VMEM((EB, CHUNK), jnp.float32),
            pltpu.VMEM((EB, CHUNK), jnp.float32),
            pltpu.VMEM_SHARED((N_PAD, CHUNK), jnp.float32),
            pltpu.SemaphoreType.DMA,
            pltpu.SemaphoreType.DMA,
            pltpu.SemaphoreType.DMA,
            pltpu.SemaphoreType.DMA,
        ],
    )(_sc_prop_body)


def _sc_prop(y2, src_p, dst_p):
    return _build_sc_prop()(
        y2,
        src_p.reshape(NS, NB2, EB),
        dst_p.reshape(NS, NB2, EB),
    )


def _sc_prop_body(y2_hbm, src_hbm, dst_hbm, tx_hbm,
                  srcv2, dstv, b0, b1, acc, gs0, gs1, ss0, ss1):
    c = lax.axis_index("c")
    s = lax.axis_index("s")

    def gather(j, buf, sem):
        return pltpu.async_copy(y2_hbm.at[srcv2.at[j]], buf, sem)

    def scatter(j, buf, sem):
        return pltpu.async_copy(buf, acc.at[dstv.at[j]], sem, add=True)

    for ch in range(NCH // NC):
        cidx = c * (NCH // NC) + ch
        off = cidx * N

        def zbody(i, _):
            for k in range(CHUNK // 16):
                b0[i, pl.ds(k * 16, 16)] = jnp.zeros((16,), jnp.float32)
            return 0

        lax.fori_loop(0, EB, zbody, 0, unroll=2)
        for t in range(NPT // EB):
            pltpu.sync_copy(b0, acc.at[pl.ds(s * NPT + t * EB, EB)])
        plsc.subcore_barrier()

        for half in range(NB2 // NBH):
            pltpu.sync_copy(src_hbm.at[s, pl.ds(half * NBH, NBH)], srcv2)
            pltpu.sync_copy(dst_hbm.at[s, pl.ds(half * NBH, NBH)], dstv)

            def obody(i, _):
                j = i // (EB // 16)
                k = i % (EB // 16)
                srcv2[j, pl.ds(k * 16, 16)] = (
                    srcv2[j, pl.ds(k * 16, 16)] + off)
                return 0

            lax.fori_loop(0, NBH * EB // 16, obody, 0, unroll=4)

            gather(0, b0, gs0)

            def pair(j2, _):
                j = j2 * 2

                @pl.when(j2 > 0)
                def _():
                    pltpu.make_async_copy(b1, acc.at[dstv.at[0]], ss1).wait()

                g1 = gather(j + 1, b1, gs1)
                pltpu.make_async_copy(y2_hbm.at[srcv2.at[0]], b0, gs0).wait()
                s0 = scatter(j, b0, ss0)
                g1.wait()
                scatter(j + 1, b1, ss1)
                s0.wait()

                @pl.when(j2 < NBH // 2 - 1)
                def _():
                    gather(j + 2, b0, gs0)

                return 0

            lax.fori_loop(0, NBH // 2, pair, 0)
            pltpu.make_async_copy(b1, acc.at[dstv.at[0]], ss1).wait()

        plsc.subcore_barrier()

        for t in range(NPT // EB):
            pltpu.sync_copy(acc.at[pl.ds(s * NPT + t * EB, EB)], b0)
            pltpu.sync_copy(
                b0, tx_hbm.at[pl.ds(cidx * N_PAD + s * NPT + t * EB, EB)])
        plsc.subcore_barrier()


def _dinv_body(deg_ref, dinv_ref):
    a = deg_ref[...]
    d = a[0:1, :] + a[1:2, :]
    dinv_ref[...] = jnp.where(d > 0, lax.rsqrt(jnp.maximum(d, 1e-12)), 0.0)


def _tc_dinv(deg):
    return pl.pallas_call(
        _dinv_body,
        out_shape=jax.ShapeDtypeStruct((1, N_PAD), jnp.float32),
    )(deg)


def _mm_body(x_ref, w_ref, o_ref):
    o_ref[...] = jnp.dot(x_ref[...], w_ref[...],
                         preferred_element_type=jnp.float32)


def _tc_matmul(x, w):
    k = x.shape[1]
    return pl.pallas_call(
        _mm_body,
        grid=(N // MB,),
        in_specs=[
            pl.BlockSpec((MB, k), lambda i: (i, 0)),
            pl.BlockSpec((k, F_H), lambda i: (0, 0)),
        ],
        out_specs=pl.BlockSpec((MB, F_H), lambda i: (i, 0)),
        out_shape=jax.ShapeDtypeStruct((N, F_H), jnp.float32),
    )(x, w)


def _mmc_body(x_ref, w_ref, dinv_ref, o_ref):
    o_ref[...] = (jnp.dot(x_ref[...], w_ref[...],
                          preferred_element_type=jnp.float32)
                  * dinv_ref[...])[None]


def _tc_matmul_chunked(x, w, dinv_col):
    k = x.shape[1]
    return pl.pallas_call(
        _mmc_body,
        grid=(N // MB, NCH),
        in_specs=[
            pl.BlockSpec((MB, k), lambda i, j: (i, 0)),
            pl.BlockSpec((k, CHUNK), lambda i, j: (0, j)),
            pl.BlockSpec((MB, 1), lambda i, j: (i, 0)),
        ],
        out_specs=pl.BlockSpec((1, MB, CHUNK), lambda i, j: (j, i, 0)),
        out_shape=jax.ShapeDtypeStruct((NCH, N, CHUNK), jnp.float32),
    )(x, w, dinv_col)


def _relu_body(xw0_ref, tx_ref, b_ref, dinv_ref, o_ref):
    t = tx_ref[...]
    tcat = jnp.concatenate([t[0], t[1], t[2], t[3]], axis=-1)
    o_ref[...] = jnp.maximum(
        xw0_ref[...] - dinv_ref[...] * tcat + b_ref[...], 0.0)


def _tc_add_relu(xw0, tx, b, dinv_col):
    return pl.pallas_call(
        _relu_body,
        grid=(N // MB,),
        in_specs=[
            pl.BlockSpec((MB, F_H), lambda i: (i, 0)),
            pl.BlockSpec((NCH, MB, CHUNK), lambda i: (0, i, 0)),
            pl.BlockSpec((1, F_H), lambda i: (0, 0)),
            pl.BlockSpec((MB, 1), lambda i: (i, 0)),
        ],
        out_specs=pl.BlockSpec((MB, F_H), lambda i: (i, 0)),
        out_shape=jax.ShapeDtypeStruct((N, F_H), jnp.float32),
    )(xw0, tx, b, dinv_col)


def _pool_body(h_ref, btr_ref, btc_ref, gmp_ref, gsum_ref, cnt_ref, m_ref):
    i = pl.program_id(0)
    h2 = h_ref[...]
    btr = btr_ref[0]
    btc = btc_ref[...]

    gids = lax.broadcasted_iota(jnp.int32, (G, MB), 0)
    m = (gids == jnp.broadcast_to(btr, (G, MB))).astype(jnp.float32)

    gids_t = lax.broadcasted_iota(jnp.int32, (MB, G), 1)
    m_ref[...] = (jnp.broadcast_to(btc, (MB, G)) == gids_t).astype(jnp.float32)

    bsum = jnp.dot(m, h2, preferred_element_type=jnp.float32)
    bcnt = jnp.sum(m, axis=1, keepdims=True)

    def mbody(k, acc):
        hblk = h_ref[pl.ds(k * 8, 8), :]
        mblk = m_ref[pl.ds(k * 8, 8), :]
        w = jnp.where(mblk[:, :, None] > 0, hblk[:, None, :], -3e38)
        return jnp.maximum(acc, jnp.max(w, axis=0))

    bmax = lax.fori_loop(0, MB // 8, mbody,
                         jnp.full((G, F_H), -3e38, jnp.float32))

    @pl.when(i == 0)
    def _():
        gmp_ref[...] = jnp.full((G, F_H), -3e38, jnp.float32)
        gsum_ref[...] = jnp.zeros((G, F_H), jnp.float32)
        cnt_ref[...] = jnp.zeros((G, 1), jnp.float32)

    gmp_ref[...] = jnp.maximum(gmp_ref[...], bmax)
    gsum_ref[...] = gsum_ref[...] + bsum
    cnt_ref[...] = cnt_ref[...] + bcnt


def _tc_pool(h2, batch_row3, batch_col):
    return pl.pallas_call(
        _pool_body,
        grid=(N // MB,),
        in_specs=[
            pl.BlockSpec((MB, F_H), lambda i: (i, 0)),
            pl.BlockSpec((1, 1, MB), lambda i: (i, 0, 0)),
            pl.BlockSpec((MB, 1), lambda i: (i, 0)),
        ],
        out_specs=[
            pl.BlockSpec((G, F_H), lambda i: (0, 0)),
            pl.BlockSpec((G, F_H), lambda i: (0, 0)),
            pl.BlockSpec((G, 1), lambda i: (0, 0)),
        ],
        out_shape=[
            jax.ShapeDtypeStruct((G, F_H), jnp.float32),
            jax.ShapeDtypeStruct((G, F_H), jnp.float32),
            jax.ShapeDtypeStruct((G, 1), jnp.float32),
        ],
        scratch_shapes=[pltpu.VMEM((MB, G), jnp.float32)],
    )(h2, batch_row3, batch_col)


def _softmax(x):
    e = jnp.exp(x - jnp.max(x, axis=-1, keepdims=True))
    return e / jnp.sum(e, axis=-1, keepdims=True)


def _heads_body(gmp_ref, gsum_ref, cnt_ref,
                w01_ref, b01_ref, w02_ref, b02_ref, w03_ref, b03_ref,
                w11_ref, b11_ref, w12_ref, b12_ref, w13_ref, b13_ref,
                out0_ref, out1_ref, feat0_ref, feat1_ref):
    cnt = cnt_ref[...]
    gap = gsum_ref[...] / jnp.maximum(cnt, 1.0)
    gmp = jnp.where(cnt > 0, gmp_ref[...], 0.0)
    feat = jnp.concatenate([gmp, gap], axis=1)
    feat0_ref[...] = feat

    def ff(v, w_ref, b_ref):
        return jnp.maximum(
            jnp.dot(v, w_ref[...], preferred_element_type=jnp.float32)
            + b_ref[...], 0.0)

    x0 = ff(feat, w01_ref, b01_ref)
    x0 = ff(x0, w02_ref, b02_ref)
    x0 = ff(x0, w03_ref, b03_ref)
    out0_ref[...] = _softmax(x0)

    x1 = ff(feat, w11_ref, b11_ref)
    x1 = ff(x1, w12_ref, b12_ref)
    feat1_ref[...] = x1
    x1 = ff(x1, w13_ref, b13_ref)
    out1_ref[...] = _softmax(x1)


def _tc_heads(gmp, gsum, cnt, w01, b01, w02, b02, w03, b03,
              w11, b11, w12, b12, w13, b13):
    return pl.pallas_call(
        _heads_body,
        out_shape=[
            jax.ShapeDtypeStruct((G, 2), jnp.float32),
            jax.ShapeDtypeStruct((G, 4), jnp.float32),
            jax.ShapeDtypeStruct((G, 1024), jnp.float32),
            jax.ShapeDtypeStruct((G, 256), jnp.float32),
        ],
    )(gmp, gsum, cnt, w01, b01, w02, b02, w03, b03,
      w11, b11, w12, b12, w13, b13)


def kernel(x, edge_index, batch, flag,
           conv1_W0, conv1_W1, conv1_b, conv2_W0, conv2_W1, conv2_b,
           lin01_W, lin01_b, lin02_W, lin02_b, lin03_W, lin03_b,
           lin11_W, lin11_b, lin12_W, lin12_b, lin13_W, lin13_b):
    src = edge_index[0].astype(jnp.int32)
    dst = edge_index[1].astype(jnp.int32)
    src_p = jnp.zeros((E_PAD,), jnp.int32).at[:E].set(src).reshape(NW, NB, EB)
    dst_p = jnp.full((E_PAD,), N, jnp.int32).at[:E].set(dst).reshape(
        NW, NB, EB)

    batch_i = batch.astype(jnp.int32)
    batch_row3 = batch_i.reshape(N // MB, 1, MB)
    batch_col = batch_i.reshape(N, 1)

    deg = _sc_deg(dst_p)
    dinv = _tc_dinv(deg)
    dinv_col = dinv.reshape(N_PAD, 1)[:N]

    xw0 = _tc_matmul(x, conv1_W0)
    xw1c = _tc_matmul_chunked(x, conv1_W1, dinv_col).reshape(NCH * N, CHUNK)
    tx1 = _sc_prop(xw1c, src_p, dst_p).reshape(NCH, N_PAD, CHUNK)
    h1 = _tc_add_relu(xw0, tx1[:, :N, :], conv1_b.reshape(1, F_H), dinv_col)

    hw0 = _tc_matmul(h1, conv2_W0)
    hw1c = _tc_matmul_chunked(h1, conv2_W1, dinv_col).reshape(NCH * N, CHUNK)
    tx2 = _sc_prop(hw1c, src_p, dst_p).reshape(NCH, N_PAD, CHUNK)
    h2 = _tc_add_relu(hw0, tx2[:, :N, :], conv2_b.reshape(1, F_H), dinv_col)

    gmp, gsum, cnt = _tc_pool(h2, batch_row3, batch_col)
    out0, out1, feat0, feat1 = _tc_heads(
        gmp, gsum, cnt,
        lin01_W, lin01_b.reshape(1, F_H), lin02_W, lin02_b.reshape(1, 256),
        lin03_W, lin03_b.reshape(1, 2),
        lin11_W, lin11_b.reshape(1, F_H), lin12_W, lin12_b.reshape(1, 256),
        lin13_W, lin13_b.reshape(1, 4))
    return (out0, out1, feat0, feat1)


# speedup vs baseline: 4.2534x; 1.0108x over previous
import functools

import jax
import jax.numpy as jnp
from jax import lax
from jax.experimental import pallas as pl
from jax.experimental.pallas import tpu as pltpu
from jax.experimental.pallas import tpu_sc as plsc

N = 10000
E = 160000
G = 64
F_H = 512
NC = 2
NS = 16
NW = NC * NS
EB = 128
NB = 40
E_PAD = NW * NB * EB
EPT = NB * EB
NB2 = 80
NBH = 40
N_PAD = 10240
NPT = N_PAD // NS
CHUNK = 128
NCH = F_H // CHUNK
MB = 1000

@functools.cache
def _build_sc_deg():
    return functools.partial(
        pl.kernel,
        out_type=jax.ShapeDtypeStruct((NC, N_PAD), jnp.float32),
        mesh=plsc.VectorSubcoreMesh(core_axis_name="c", subcore_axis_name="s"),
        compiler_params=pltpu.CompilerParams(needs_layout_passes=False),
        scratch_types=[
            pltpu.VMEM((NB, EB), jnp.int32),
            pltpu.VMEM((N_PAD,), jnp.float32),
            pltpu.VMEM((NS, NPT), jnp.float32),
            pltpu.VMEM((NPT,), jnp.float32),
            pltpu.VMEM_SHARED((NS, N_PAD), jnp.float32),
        ],
    )(_sc_deg_body)


def _sc_deg(dst_p):
    return _build_sc_deg()(dst_p)


def _sc_deg_body(dst_hbm, deg_hbm, dstv, histv, colbuf, degv, part):
    c = lax.axis_index("c")
    s = lax.axis_index("s")
    wid = c * NS + s
    pltpu.sync_copy(dst_hbm.at[wid], dstv)

    def zbody(i, _):
        histv[pl.ds(i * 16, 16)] = jnp.zeros((16,), jnp.float32)
        return 0

    lax.fori_loop(0, N_PAD // 16, zbody, 0, unroll=4)

    ones = jnp.ones((16,), jnp.float32)
    lanes = lax.iota(jnp.int32, 16)
    base = wid * EPT

    def ebody(i, _):
        j = i // (EB // 16)
        k = i % (EB // 16)
        idx = dstv[j, pl.ds(k * 16, 16)]
        gid = base + i * 16 + lanes
        plsc.addupdate_scatter(histv, [idx], ones, mask=gid < E)
        return 0

    lax.fori_loop(0, EPT // 16, ebody, 0, unroll=4)

    pltpu.sync_copy(histv, part.at[s])
    plsc.subcore_barrier()
    pltpu.sync_copy(part.at[:, pl.ds(s * NPT, NPT)], colbuf)

    def sbody(i, _):
        acc = colbuf[0, pl.ds(i * 16, 16)]
        for j in range(1, NS):
            acc = acc + colbuf[j, pl.ds(i * 16, 16)]
        degv[pl.ds(i * 16, 16)] = acc
        return 0

    lax.fori_loop(0, NPT // 16, sbody, 0, unroll=2)
    pltpu.sync_copy(degv, deg_hbm.at[c, pl.ds(s * NPT, NPT)])


@functools.cache
def _build_sc_prop():
    return functools.partial(
        pl.kernel,
        out_type=jax.ShapeDtypeStruct((NCH * N_PAD, CHUNK), jnp.float32),
        mesh=plsc.VectorSubcoreMesh(core_axis_name="c", subcore_axis_name="s"),
        compiler_params=pltpu.CompilerParams(needs_layout_passes=False),
        scratch_types=[
            pltpu.VMEM((NBH, EB), jnp.int32),
            pltpu.VMEM((NBH, EB), jnp.int32),
            pltpu.VMEM((EB, CHUNK), jnp.float32),
            pltpu.VMEM((EB, CHUNK), jnp.float32),
            pltpu.VMEM_SHARED((N_PAD, CHUNK), jnp.float32),
            pltpu.SemaphoreType.DMA,
            pltpu.SemaphoreType.DMA,
            pltpu.SemaphoreType.DMA,
            pltpu.SemaphoreType.DMA,
        ],
    )(_sc_prop_body)


def _sc_prop(y2, src_p, dst_p):
    return _build_sc_prop()(
        y2,
        src_p.reshape(NS, NB2, EB),
        dst_p.reshape(NS, NB2, EB),
    )


def _sc_prop_body(y2_hbm, src_hbm, dst_hbm, tx_hbm,
                  srcv2, dstv, b0, b1, acc, gs0, gs1, ss0, ss1):
    c = lax.axis_index("c")
    s = lax.axis_index("s")

    def gather(j, buf, sem):
        return pltpu.async_copy(y2_hbm.at[srcv2.at[j]], buf, sem)

    def scatter(j, buf, sem):
        return pltpu.async_copy(buf, acc.at[dstv.at[j]], sem, add=True)

    for ch in range(NCH // NC):
        cidx = c * (NCH // NC) + ch
        off = cidx * N

        def zbody(i, _):
            for k in range(CHUNK // 16):
                b0[i, pl.ds(k * 16, 16)] = jnp.zeros((16,), jnp.float32)
            return 0

        lax.fori_loop(0, EB, zbody, 0, unroll=2)
        for t in range(NPT // EB):
            pltpu.sync_copy(b0, acc.at[pl.ds(s * NPT + t * EB, EB)])
        plsc.subcore_barrier()

        for half in range(NB2 // NBH):
            pltpu.sync_copy(src_hbm.at[s, pl.ds(half * NBH, NBH)], srcv2)
            pltpu.sync_copy(dst_hbm.at[s, pl.ds(half * NBH, NBH)], dstv)

            def obody(i, _):
                j = i // (EB // 16)
                k = i % (EB // 16)
                srcv2[j, pl.ds(k * 16, 16)] = (
                    srcv2[j, pl.ds(k * 16, 16)] + off)
                return 0

            lax.fori_loop(0, NBH * EB // 16, obody, 0, unroll=4)

            gather(0, b0, gs0)

            def pair(j2, _):
                j = j2 * 2

                @pl.when(j2 > 0)
                def _():
                    pltpu.make_async_copy(b1, acc.at[dstv.at[0]], ss1).wait()

                g1 = gather(j + 1, b1, gs1)
                pltpu.make_async_copy(y2_hbm.at[srcv2.at[0]], b0, gs0).wait()
                s0 = scatter(j, b0, ss0)
                g1.wait()
                scatter(j + 1, b1, ss1)
                s0.wait()

                @pl.when(j2 < NBH // 2 - 1)
                def _():
                    gather(j + 2, b0, gs0)

                return 0

            lax.fori_loop(0, NBH // 2, pair, 0)
            pltpu.make_async_copy(b1, acc.at[dstv.at[0]], ss1).wait()

        plsc.subcore_barrier()

        for t in range(NPT // EB):
            pltpu.sync_copy(acc.at[pl.ds(s * NPT + t * EB, EB)], b0)
            pltpu.sync_copy(
                b0, tx_hbm.at[pl.ds(cidx * N_PAD + s * NPT + t * EB, EB)])
        plsc.subcore_barrier()


def _dinv_body(deg_ref, dinv_ref):
    a = deg_ref[...]
    d = a[0:1, :] + a[1:2, :]
    dinv_ref[...] = jnp.where(d > 0, lax.rsqrt(jnp.maximum(d, 1e-12)), 0.0)


def _tc_dinv(deg):
    return pl.pallas_call(
        _dinv_body,
        out_shape=jax.ShapeDtypeStruct((1, N_PAD), jnp.float32),
    )(deg)


def _mm_body(x_ref, w_ref, o_ref):
    o_ref[...] = jnp.dot(x_ref[...], w_ref[...],
                         preferred_element_type=jnp.float32)


def _tc_matmul(x, w):
    k = x.shape[1]
    return pl.pallas_call(
        _mm_body,
        grid=(N // MB,),
        in_specs=[
            pl.BlockSpec((MB, k), lambda i: (i, 0)),
            pl.BlockSpec((k, F_H), lambda i: (0, 0)),
        ],
        out_specs=pl.BlockSpec((MB, F_H), lambda i: (i, 0)),
        out_shape=jax.ShapeDtypeStruct((N, F_H), jnp.float32),
    )(x, w)


def _mmc_body(x_ref, w_ref, dinv_ref, o_ref):
    o_ref[...] = (jnp.dot(x_ref[...], w_ref[...],
                          preferred_element_type=jnp.float32)
                  * dinv_ref[...])[None]


def _tc_matmul_chunked(x, w, dinv_col):
    k = x.shape[1]
    return pl.pallas_call(
        _mmc_body,
        grid=(N // MB, NCH),
        in_specs=[
            pl.BlockSpec((MB, k), lambda i, j: (i, 0)),
            pl.BlockSpec((k, CHUNK), lambda i, j: (0, j)),
            pl.BlockSpec((MB, 1), lambda i, j: (i, 0)),
        ],
        out_specs=pl.BlockSpec((1, MB, CHUNK), lambda i, j: (j, i, 0)),
        out_shape=jax.ShapeDtypeStruct((NCH, N, CHUNK), jnp.float32),
    )(x, w, dinv_col)


def _relu_body(xw0_ref, tx_ref, b_ref, dinv_ref, o_ref):
    t = tx_ref[...]
    tcat = jnp.concatenate([t[0], t[1], t[2], t[3]], axis=-1)
    o_ref[...] = jnp.maximum(
        xw0_ref[...] - dinv_ref[...] * tcat + b_ref[...], 0.0)


def _tc_add_relu(xw0, tx, b, dinv_col):
    return pl.pallas_call(
        _relu_body,
        grid=(N // MB,),
        in_specs=[
            pl.BlockSpec((MB, F_H), lambda i: (i, 0)),
            pl.BlockSpec((NCH, MB, CHUNK), lambda i: (0, i, 0)),
            pl.BlockSpec((1, F_H), lambda i: (0, 0)),
            pl.BlockSpec((MB, 1), lambda i: (i, 0)),
        ],
        out_specs=pl.BlockSpec((MB, F_H), lambda i: (i, 0)),
        out_shape=jax.ShapeDtypeStruct((N, F_H), jnp.float32),
    )(xw0, tx, b, dinv_col)


def _pool_body(h_ref, btr_ref, btc_ref, gmp_ref, gsum_ref, cnt_ref, m_ref):
    i = pl.program_id(0)
    h2 = h_ref[...]
    btr = btr_ref[0]
    btc = btc_ref[...]

    gids = lax.broadcasted_iota(jnp.int32, (G, MB), 0)
    m = (gids == jnp.broadcast_to(btr, (G, MB))).astype(jnp.float32)

    gids_t = lax.broadcasted_iota(jnp.int32, (MB, G), 1)
    m_ref[...] = (jnp.broadcast_to(btc, (MB, G)) == gids_t).astype(jnp.float32)

    bsum = jnp.dot(m, h2, preferred_element_type=jnp.float32)
    bcnt = jnp.sum(m, axis=1, keepdims=True)

    def mbody(k, acc):
        hblk = h_ref[pl.ds(k * 8, 8), :]
        mblk = m_ref[pl.ds(k * 8, 8), :]
        w = jnp.where(mblk[:, :, None] > 0, hblk[:, None, :], -3e38)
        return jnp.maximum(acc, jnp.max(w, axis=0))

    bmax = lax.fori_loop(0, MB // 8, mbody,
                         jnp.full((G, F_H), -3e38, jnp.float32))

    @pl.when(i == 0)
    def _():
        gmp_ref[...] = jnp.full((G, F_H), -3e38, jnp.float32)
        gsum_ref[...] = jnp.zeros((G, F_H), jnp.float32)
        cnt_ref[...] = jnp.zeros((G, 1), jnp.float32)

    gmp_ref[...] = jnp.maximum(gmp_ref[...], bmax)
    gsum_ref[...] = gsum_ref[...] + bsum
    cnt_ref[...] = cnt_ref[...] + bcnt


def _tc_pool(h2, batch_row3, batch_col):
    return pl.pallas_call(
        _pool_body,
        grid=(N // MB,),
        in_specs=[
            pl.BlockSpec((MB, F_H), lambda i: (i, 0)),
            pl.BlockSpec((1, 1, MB), lambda i: (i, 0, 0)),
            pl.BlockSpec((MB, 1), lambda i: (i, 0)),
        ],
        out_specs=[
            pl.BlockSpec((G, F_H), lambda i: (0, 0)),
            pl.BlockSpec((G, F_H), lambda i: (0, 0)),
            pl.BlockSpec((G, 1), lambda i: (0, 0)),
        ],
        out_shape=[
            jax.ShapeDtypeStruct((G, F_H), jnp.float32),
            jax.ShapeDtypeStruct((G, F_H), jnp.float32),
            jax.ShapeDtypeStruct((G, 1), jnp.float32),
        ],
        scratch_shapes=[pltpu.VMEM((MB, G), jnp.float32)],
    )(h2, batch_row3, batch_col)


def _softmax(x):
    e = jnp.exp(x - jnp.max(x, axis=-1, keepdims=True))
    return e / jnp.sum(e, axis=-1, keepdims=True)


def _heads_body(gmp_ref, gsum_ref, cnt_ref,
                w01_ref, b01_ref, w02_ref, b02_ref, w03_ref, b03_ref,
                w11_ref, b11_ref, w12_ref, b12_ref, w13_ref, b13_ref,
                out0_ref, out1_ref, feat0_ref, feat1_ref):
    cnt = cnt_ref[...]
    gap = gsum_ref[...] / jnp.maximum(cnt, 1.0)
    gmp = jnp.where(cnt > 0, gmp_ref[...], 0.0)
    feat = jnp.concatenate([gmp, gap], axis=1)
    feat0_ref[...] = feat

    def ff(v, w_ref, b_ref):
        return jnp.maximum(
            jnp.dot(v, w_ref[...], preferred_element_type=jnp.float32)
            + b_ref[...], 0.0)

    x0 = ff(feat, w01_ref, b01_ref)
    x0 = ff(x0, w02_ref, b02_ref)
    x0 = ff(x0, w03_ref, b03_ref)
    out0_ref[...] = _softmax(x0)

    x1 = ff(feat, w11_ref, b11_ref)
    x1 = ff(x1, w12_ref, b12_ref)
    feat1_ref[...] = x1
    x1 = ff(x1, w13_ref, b13_ref)
    out1_ref[...] = _softmax(x1)


def _tc_heads(gmp, gsum, cnt, w01, b01, w02, b02, w03, b03,
              w11, b11, w12, b12, w13, b13):
    return pl.pallas_call(
        _heads_body,
        out_shape=[
            jax.ShapeDtypeStruct((G, 2), jnp.float32),
            jax.ShapeDtypeStruct((G, 4), jnp.float32),
            jax.ShapeDtypeStruct((G, 1024), jnp.float32),
            jax.ShapeDtypeStruct((G, 256), jnp.float32),
        ],
    )(gmp, gsum, cnt, w01, b01, w02, b02, w03, b03,
      w11, b11, w12, b12, w13, b13)


def kernel(x, edge_index, batch, flag,
           conv1_W0, conv1_W1, conv1_b, conv2_W0, conv2_W1, conv2_b,
           lin01_W, lin01_b, lin02_W, lin02_b, lin03_W, lin03_b,
           lin11_W, lin11_b, lin12_W, lin12_b, lin13_W, lin13_b):
    src = edge_index[0].astype(jnp.int32)
    dst = edge_index[1].astype(jnp.int32)
    src_p = jnp.zeros((E_PAD,), jnp.int32).at[:E].set(src).reshape(NW, NB, EB)
    dst_p = jnp.full((E_PAD,), N, jnp.int32).at[:E].set(dst).reshape(
        NW, NB, EB)

    batch_i = batch.astype(jnp.int32)
    batch_row3 = batch_i.reshape(N // MB, 1, MB)
    batch_col = batch_i.reshape(N, 1)

    deg = _sc_deg(dst_p)
    dinv = _tc_dinv(deg)
    dinv_col = dinv.reshape(N_PAD, 1)[:N]

    xb = x.astype(jnp.bfloat16)
    xw0 = _tc_matmul(xb, conv1_W0.astype(jnp.bfloat16))
    xw1c = _tc_matmul_chunked(
        xb, conv1_W1.astype(jnp.bfloat16), dinv_col).reshape(NCH * N, CHUNK)
    tx1 = _sc_prop(xw1c, src_p, dst_p).reshape(NCH, N_PAD, CHUNK)
    h1 = _tc_add_relu(xw0, tx1[:, :N, :], conv1_b.reshape(1, F_H), dinv_col)

    h1b = h1.astype(jnp.bfloat16)
    hw0 = _tc_matmul(h1b, conv2_W0.astype(jnp.bfloat16))
    hw1c = _tc_matmul_chunked(
        h1b, conv2_W1.astype(jnp.bfloat16), dinv_col).reshape(NCH * N, CHUNK)
    tx2 = _sc_prop(hw1c, src_p, dst_p).reshape(NCH, N_PAD, CHUNK)
    h2 = _tc_add_relu(hw0, tx2[:, :N, :], conv2_b.reshape(1, F_H), dinv_col)

    gmp, gsum, cnt = _tc_pool(h2, batch_row3, batch_col)
    out0, out1, feat0, feat1 = _tc_heads(
        gmp, gsum, cnt,
        lin01_W, lin01_b.reshape(1, F_H), lin02_W, lin02_b.reshape(1, 256),
        lin03_W, lin03_b.reshape(1, 2),
        lin11_W, lin11_b.reshape(1, F_H), lin12_W, lin12_b.reshape(1, 256),
        lin13_W, lin13_b.reshape(1, 4))
    return (out0, out1, feat0, feat1)
